# baseline (speedup 1.0000x reference)
import jax
import jax.numpy as jnp
import numpy as np
from jax.experimental import pallas as pl
from jax.experimental.pallas import tpu as pltpu

_axis = np.arange(5, dtype=np.float64) - 2
_g = np.exp(-(_axis ** 2) / 2.0)
_g = _g / _g.sum()
_G0, _G1, _G2 = float(_g[2]), float(_g[1]), float(_g[0])

_T1 = float(np.tan(np.radians(22.5)))
_T2 = float(np.tan(np.radians(67.5)))

_PAD = 8


def _canny_body(x_ref, y_ref, o_ref):
    h, w = x_ref.shape[2], x_ref.shape[3]
    hp = h + 2 * _PAD
    row_i = jax.lax.broadcasted_iota(jnp.int32, (hp, w), 0)
    col_i = jax.lax.broadcasted_iota(jnp.int32, (hp, w), 1)
    col_p = col_i < (w - 1)
    col_m = col_i >= 1
    row_ok = (row_i >= _PAD) & (row_i < h + _PAD)
    zero = jnp.float32(0.0)
    zpad = jnp.zeros((_PAD, w), jnp.float32)

    def sc_p(a):
        return jnp.where(col_p, jnp.roll(a, -1, 1), zero)

    def sc_m(a):
        return jnp.where(col_m, jnp.roll(a, 1, 1), zero)

    def sr_p(a):
        return jnp.roll(a, -1, 0)

    def sr_m(a):
        return jnp.roll(a, 1, 0)

    def canny2d(img):
        p = jnp.concatenate([zpad, img, zpad], axis=0)

        cp1 = sc_p(p)
        cm1 = sc_m(p)
        r = _G0 * p + _G1 * (cp1 + cm1) + _G2 * (sc_p(cp1) + sc_m(cm1))
        rp1 = sr_p(r)
        rm1 = sr_m(r)
        blur = _G0 * r + _G1 * (rp1 + rm1) + _G2 * (sr_p(rp1) + sr_m(rm1))
        blur = jnp.where(row_ok, blur, zero)

        bp = sc_p(blur)
        bm = sc_m(blur)
        rd = bp - bm
        rs = bp + 2.0 * blur + bm
        gx = sr_p(rd) + 2.0 * rd + sr_m(rd)
        gy = sr_p(rs) - sr_m(rs)

        mag = jnp.where(row_ok, jnp.sqrt(gx * gx + gy * gy + 1e-12), zero)

        ax_ = jnp.abs(gx)
        ay_ = jnp.abs(gy)
        is_h = ay_ < _T1 * ax_
        is_v = ay_ > _T2 * ax_
        gx_pos = gx > zero
        same_q = gx * gy > zero
        m = jnp.where(is_h, jnp.where(gx_pos, 0.0, 180.0),
                      jnp.where(is_v, 90.0, jnp.where(gx_pos, 45.0, 135.0)))
        ori = 180.0 + jnp.sign(gy) * m

        mcp = sc_p(mag)
        mcm = sc_m(mag)
        nb0 = jnp.maximum(mcp, mcm)
        nb1 = jnp.maximum(sr_m(mcp), sr_p(mcm))
        nb2 = jnp.maximum(sr_m(mag), sr_p(mag))
        nb3 = jnp.maximum(sr_m(mcm), sr_p(mcp))
        nb = jnp.where(is_h, nb0,
                       jnp.where(is_v, nb2, jnp.where(same_q, nb1, nb3)))
        thin = jnp.where(mag > nb, mag, zero)
        return thin, ori

    for k in range(x_ref.shape[0]):
        tx, ox = canny2d(x_ref[k, 0])
        ty, oy = canny2d(y_ref[k, 0])
        d1 = jnp.abs(tx[_PAD:h + _PAD] - ty[_PAD:h + _PAD])
        d2 = jnp.abs(ox[_PAD:h + _PAD] - oy[_PAD:h + _PAD])
        s1 = jnp.sum(d1, axis=0, keepdims=True)
        s2 = jnp.sum(d2, axis=0, keepdims=True)
        o_ref[k] = jnp.concatenate([s1, s2], axis=0)


@jax.jit
def kernel(X, Y):
    b, _, h, w = X.shape
    bb = 4 if b % 4 == 0 else 1
    out = pl.pallas_call(
        _canny_body,
        grid=(b // bb,),
        in_specs=[
            pl.BlockSpec((bb, 1, h, w), lambda i: (i, 0, 0, 0)),
            pl.BlockSpec((bb, 1, h, w), lambda i: (i, 0, 0, 0)),
        ],
        out_specs=pl.BlockSpec((bb, 2, w), lambda i: (i, 0, 0)),
        out_shape=jax.ShapeDtypeStruct((b, 2, w), jnp.float32),
        compiler_params=pltpu.CompilerParams(
            dimension_semantics=("arbitrary",),
        ),
    )(X, Y)
    return out.sum() / jnp.float32(h * w)


# speedup vs baseline: 1.3229x; 1.3229x over previous
import jax
import jax.numpy as jnp
import numpy as np
from jax.experimental import pallas as pl
from jax.experimental.pallas import tpu as pltpu

_axis = np.arange(5, dtype=np.float64) - 2
_g = np.exp(-(_axis ** 2) / 2.0)
_g = _g / _g.sum()
_G0, _G1, _G2 = float(_g[2]), float(_g[1]), float(_g[0])

_T1 = float(np.tan(np.radians(22.5)))
_T2 = float(np.tan(np.radians(67.5)))

_PAD = 8


def _canny_body(x_ref, y_ref, o_ref):
    h, w = x_ref.shape[2], x_ref.shape[3]
    hp = h + 2 * _PAD
    row_i = jax.lax.broadcasted_iota(jnp.int32, (hp, w), 0)
    col_i = jax.lax.broadcasted_iota(jnp.int32, (hp, w), 1)
    col_p = col_i < (w - 1)
    col_m = col_i >= 1
    row_ok = (row_i >= _PAD) & (row_i < h + _PAD)
    zero = jnp.float32(0.0)
    zpad = jnp.zeros((_PAD, w), jnp.float32)

    def sc_p(a):
        return jnp.where(col_p, jnp.roll(a, -1, 1), zero)

    def sc_m(a):
        return jnp.where(col_m, jnp.roll(a, 1, 1), zero)

    def sr_p(a):
        return jnp.roll(a, -1, 0)

    def sr_m(a):
        return jnp.roll(a, 1, 0)

    def canny2d(img):
        p = jnp.concatenate([zpad, img, zpad], axis=0)

        cp1 = sc_p(p)
        cm1 = sc_m(p)
        r = _G0 * p + _G1 * (cp1 + cm1) + _G2 * (sc_p(cp1) + sc_m(cm1))
        rp1 = sr_p(r)
        rm1 = sr_m(r)
        blur = _G0 * r + _G1 * (rp1 + rm1) + _G2 * (sr_p(rp1) + sr_m(rm1))
        blur = jnp.where(row_ok, blur, zero)

        bp = sc_p(blur)
        bm = sc_m(blur)
        rd = bp - bm
        rs = bp + 2.0 * blur + bm
        gx = sr_p(rd) + 2.0 * rd + sr_m(rd)
        gy = sr_p(rs) - sr_m(rs)

        mag = jnp.where(row_ok, jnp.sqrt(gx * gx + gy * gy + 1e-12), zero)

        ax_ = jnp.abs(gx)
        ay_ = jnp.abs(gy)
        is_h = ay_ < _T1 * ax_
        is_v = ay_ > _T2 * ax_
        gx_pos = gx > zero
        same_q = gx * gy > zero
        m = jnp.where(is_h, jnp.where(gx_pos, 0.0, 180.0),
                      jnp.where(is_v, 90.0, jnp.where(gx_pos, 45.0, 135.0)))
        ori = 180.0 + jnp.sign(gy) * m

        mcp = sc_p(mag)
        mcm = sc_m(mag)
        nb0 = jnp.maximum(mcp, mcm)
        nb1 = jnp.maximum(sr_m(mcp), sr_p(mcm))
        nb2 = jnp.maximum(sr_m(mag), sr_p(mag))
        nb3 = jnp.maximum(sr_m(mcm), sr_p(mcp))
        nb = jnp.where(is_h, nb0,
                       jnp.where(is_v, nb2, jnp.where(same_q, nb1, nb3)))
        thin = jnp.where(mag > nb, mag, zero)
        return thin, ori

    for k in range(x_ref.shape[0]):
        tx, ox = canny2d(x_ref[k, 0])
        ty, oy = canny2d(y_ref[k, 0])
        d1 = jnp.abs(tx[_PAD:h + _PAD] - ty[_PAD:h + _PAD])
        d2 = jnp.abs(ox[_PAD:h + _PAD] - oy[_PAD:h + _PAD])
        s1 = jnp.sum(d1, axis=0, keepdims=True)
        s2 = jnp.sum(d2, axis=0, keepdims=True)
        o_ref[k] = jnp.concatenate([s1, s2], axis=0)


@jax.jit
def kernel(X, Y):
    b, _, h, w = X.shape
    bb = 2 if b % 2 == 0 else 1
    out = pl.pallas_call(
        _canny_body,
        grid=(b // bb,),
        in_specs=[
            pl.BlockSpec((bb, 1, h, w), lambda i: (i, 0, 0, 0)),
            pl.BlockSpec((bb, 1, h, w), lambda i: (i, 0, 0, 0)),
        ],
        out_specs=pl.BlockSpec((bb, 2, w), lambda i: (i, 0, 0)),
        out_shape=jax.ShapeDtypeStruct((b, 2, w), jnp.float32),
        compiler_params=pltpu.CompilerParams(
            dimension_semantics=("arbitrary",),
        ),
    )(X, Y)
    return out.sum() / jnp.float32(h * w)


# speedup vs baseline: 1.3514x; 1.0215x over previous
import jax
import jax.numpy as jnp
import numpy as np
from jax.experimental import pallas as pl
from jax.experimental.pallas import tpu as pltpu

_axis = np.arange(5, dtype=np.float64) - 2
_g = np.exp(-(_axis ** 2) / 2.0)
_g = _g / _g.sum()
_G0, _G1, _G2 = float(_g[2]), float(_g[1]), float(_g[0])

_T1 = float(np.tan(np.radians(22.5)))
_T2 = float(np.tan(np.radians(67.5)))

_PAD = 8


def _canny_body(x_ref, y_ref, o_ref):
    h, w = x_ref.shape[2], x_ref.shape[3]
    hp = h + 2 * _PAD
    row_i = jax.lax.broadcasted_iota(jnp.int32, (hp, w), 0)
    col_i = jax.lax.broadcasted_iota(jnp.int32, (hp, w), 1)
    col_p = col_i < (w - 1)
    col_m = col_i >= 1
    row_ok = (row_i >= _PAD) & (row_i < h + _PAD)
    zero = jnp.float32(0.0)
    zpad = jnp.zeros((_PAD, w), jnp.float32)

    def sc_p(a):
        return jnp.where(col_p, jnp.roll(a, -1, 1), zero)

    def sc_m(a):
        return jnp.where(col_m, jnp.roll(a, 1, 1), zero)

    def sr_p(a):
        return jnp.roll(a, -1, 0)

    def sr_m(a):
        return jnp.roll(a, 1, 0)

    def canny2d(img):
        p = jnp.concatenate([zpad, img, zpad], axis=0)

        cp1 = sc_p(p)
        cm1 = sc_m(p)
        r = _G0 * p + _G1 * (cp1 + cm1) + _G2 * (sc_p(cp1) + sc_m(cm1))
        rp1 = sr_p(r)
        rm1 = sr_m(r)
        blur = _G0 * r + _G1 * (rp1 + rm1) + _G2 * (sr_p(rp1) + sr_m(rm1))
        blur = jnp.where(row_ok, blur, zero)

        bp = sc_p(blur)
        bm = sc_m(blur)
        rd = bp - bm
        rs = bp + 2.0 * blur + bm
        gx = sr_p(rd) + 2.0 * rd + sr_m(rd)
        gy = sr_p(rs) - sr_m(rs)

        mag = jnp.where(row_ok, jnp.sqrt(gx * gx + gy * gy + 1e-12), zero)

        ax_ = jnp.abs(gx)
        ay_ = jnp.abs(gy)
        is_h = ay_ < _T1 * ax_
        is_v = ay_ > _T2 * ax_
        gx_pos = gx > zero
        same_q = gx * gy > zero
        m = jnp.where(is_h, jnp.where(gx_pos, 0.0, 180.0),
                      jnp.where(is_v, 90.0, jnp.where(gx_pos, 45.0, 135.0)))
        ori = 180.0 + jnp.sign(gy) * m

        mcp = sc_p(mag)
        mcm = sc_m(mag)
        nb0 = jnp.maximum(mcp, mcm)
        nb1 = jnp.maximum(sr_m(mcp), sr_p(mcm))
        nb2 = jnp.maximum(sr_m(mag), sr_p(mag))
        nb3 = jnp.maximum(sr_m(mcm), sr_p(mcp))
        nb = jnp.where(is_h, nb0,
                       jnp.where(is_v, nb2, jnp.where(same_q, nb1, nb3)))
        thin = jnp.where(mag > nb, mag, zero)
        return thin, ori

    for k in range(x_ref.shape[0]):
        tx, ox = canny2d(x_ref[k, 0])
        ty, oy = canny2d(y_ref[k, 0])
        d1 = jnp.abs(tx[_PAD:h + _PAD] - ty[_PAD:h + _PAD])
        d2 = jnp.abs(ox[_PAD:h + _PAD] - oy[_PAD:h + _PAD])
        s1 = jnp.sum(d1, axis=0, keepdims=True)
        s2 = jnp.sum(d2, axis=0, keepdims=True)
        o_ref[k] = jnp.concatenate([s1, s2], axis=0)


@jax.jit
def kernel(X, Y):
    b, _, h, w = X.shape
    bb = 1
    out = pl.pallas_call(
        _canny_body,
        grid=(b // bb,),
        in_specs=[
            pl.BlockSpec((bb, 1, h, w), lambda i: (i, 0, 0, 0)),
            pl.BlockSpec((bb, 1, h, w), lambda i: (i, 0, 0, 0)),
        ],
        out_specs=pl.BlockSpec((bb, 2, w), lambda i: (i, 0, 0)),
        out_shape=jax.ShapeDtypeStruct((b, 2, w), jnp.float32),
        compiler_params=pltpu.CompilerParams(
            dimension_semantics=("arbitrary",),
        ),
    )(X, Y)
    return out.sum() / jnp.float32(h * w)


# speedup vs baseline: 1.3530x; 1.0012x over previous
import jax
import jax.numpy as jnp
import numpy as np
from jax.experimental import pallas as pl
from jax.experimental.pallas import tpu as pltpu

_axis = np.arange(5, dtype=np.float64) - 2
_g = np.exp(-(_axis ** 2) / 2.0)
_g = _g / _g.sum()
_G0, _G1, _G2 = float(_g[2]), float(_g[1]), float(_g[0])

_T1 = float(np.tan(np.radians(22.5)))
_T2 = float(np.tan(np.radians(67.5)))

_PAD = 8


def _canny_body(x_ref, y_ref, o_ref):
    h, w = x_ref.shape[2], x_ref.shape[3]
    hp = h + 2 * _PAD
    row_i = jax.lax.broadcasted_iota(jnp.int32, (hp, w), 0)
    col_i = jax.lax.broadcasted_iota(jnp.int32, (hp, w), 1)
    col_p = col_i < (w - 1)
    col_m = col_i >= 1
    row_ok = (row_i >= _PAD) & (row_i < h + _PAD)
    zero = jnp.float32(0.0)
    zpad = jnp.zeros((_PAD, w), jnp.float32)

    def sc_p(a):
        return jnp.where(col_p, jnp.roll(a, -1, 1), zero)

    def sc_m(a):
        return jnp.where(col_m, jnp.roll(a, 1, 1), zero)

    def sr_p(a):
        return jnp.roll(a, -1, 0)

    def sr_m(a):
        return jnp.roll(a, 1, 0)

    def canny2d(img):
        p = jnp.concatenate([zpad, img, zpad], axis=0)

        cp1 = sc_p(p)
        cm1 = sc_m(p)
        r = _G0 * p + _G1 * (cp1 + cm1) + _G2 * (sc_p(cp1) + sc_m(cm1))
        rp1 = sr_p(r)
        rm1 = sr_m(r)
        blur = _G0 * r + _G1 * (rp1 + rm1) + _G2 * (sr_p(rp1) + sr_m(rm1))
        blur = jnp.where(row_ok, blur, zero)

        bp = sc_p(blur)
        bm = sc_m(blur)
        rd = bp - bm
        rs = bp + 2.0 * blur + bm
        gx = sr_p(rd) + 2.0 * rd + sr_m(rd)
        gy = sr_p(rs) - sr_m(rs)

        mag = jnp.where(row_ok, jnp.sqrt(gx * gx + gy * gy + 1e-12), zero)

        ax_ = jnp.abs(gx)
        ay_ = jnp.abs(gy)
        is_h = ay_ < _T1 * ax_
        is_v = ay_ > _T2 * ax_
        gx_pos = gx > zero
        same_q = gx * gy > zero
        m = jnp.where(is_h, jnp.where(gx_pos, 0.0, 180.0),
                      jnp.where(is_v, 90.0, jnp.where(gx_pos, 45.0, 135.0)))
        ori = 180.0 + jnp.sign(gy) * m

        mcp = sc_p(mag)
        mcm = sc_m(mag)
        nb0 = jnp.maximum(mcp, mcm)
        nb1 = jnp.maximum(sr_m(mcp), sr_p(mcm))
        nb2 = jnp.maximum(sr_m(mag), sr_p(mag))
        nb3 = jnp.maximum(sr_m(mcm), sr_p(mcp))
        nb = jnp.where(is_h, nb0,
                       jnp.where(is_v, nb2, jnp.where(same_q, nb1, nb3)))
        thin = jnp.where(mag > nb, mag, zero)
        return thin, ori

    tx, ox = canny2d(x_ref[0, 0])
    ty, oy = canny2d(y_ref[0, 0])
    d1 = jnp.abs(tx[_PAD:h + _PAD] - ty[_PAD:h + _PAD])
    d2 = jnp.abs(ox[_PAD:h + _PAD] - oy[_PAD:h + _PAD])
    s1 = jnp.sum(d1, axis=0, keepdims=True)
    s2 = jnp.sum(d2, axis=0, keepdims=True)
    cur = jnp.concatenate([s1, s2], axis=0)

    @pl.when(pl.program_id(0) == 0)
    def _init():
        o_ref[0] = cur

    @pl.when(pl.program_id(0) > 0)
    def _acc():
        o_ref[0] = o_ref[0] + cur


@jax.jit
def kernel(X, Y):
    b, _, h, w = X.shape
    out = pl.pallas_call(
        _canny_body,
        grid=(b,),
        in_specs=[
            pl.BlockSpec((1, 1, h, w), lambda i: (i, 0, 0, 0)),
            pl.BlockSpec((1, 1, h, w), lambda i: (i, 0, 0, 0)),
        ],
        out_specs=pl.BlockSpec((1, 2, w), lambda i: (0, 0, 0)),
        out_shape=jax.ShapeDtypeStruct((1, 2, w), jnp.float32),
        compiler_params=pltpu.CompilerParams(
            dimension_semantics=("arbitrary",),
        ),
    )(X, Y)
    return out.sum() / jnp.float32(h * w)
